# Initial kernel scaffold; baseline (speedup 1.0000x reference)
import jax
import jax.numpy as jnp
from jax.experimental import pallas as pl


def kernel(x, w1bd, w2m, b2m, wc, rf, blm):
    raise NotImplementedError("write your pallas kernel here")


# speedup vs baseline: 1.6345x; 1.6345x over previous
import jax
import jax.numpy as jnp
from jax import lax
from jax.experimental import pallas as pl
from jax.experimental.pallas import tpu as pltpu


_S1 = 336


def _fused_body(G, SG, C2G):
    def body(p1_ref, w1_ref, w2_ref, b2_ref, wc_ref, rf_ref, bl_ref,
             o_ref, a1_ref, c2_ref):
        for r0 in range(0, SG, _S1):
            z = jnp.dot(p1_ref[r0:r0 + _S1, :], w1_ref[...],
                        preferred_element_type=jnp.float32)
            m = jnp.maximum(jnp.maximum(z[:, 0:16], z[:, 16:32]),
                            jnp.maximum(z[:, 32:48], z[:, 48:64]))
            a1_ref[r0:r0 + _S1, :] = jnp.maximum(m, 0.0).astype(jnp.bfloat16)

        for r0 in range(0, C2G, 256):
            rs = min(256, C2G - r0)
            taps = [a1_ref[r0 + 18 * kh + kw:r0 + 18 * kh + kw + rs, :]
                    for kh in range(5) for kw in range(5)]
            col = jnp.concatenate(taps, axis=1)
            c2_ref[r0:r0 + rs, :] = jnp.dot(col, w2_ref[...],
                                            preferred_element_type=jnp.float32)

        blocks = []
        for g in range(G):
            for i in range(7):
                b0 = _S1 * g + 36 * i
                pm = jnp.maximum(
                    jnp.maximum(c2_ref[pl.ds(b0, 7, stride=2), :],
                                c2_ref[pl.ds(b0 + 1, 7, stride=2), :]),
                    jnp.maximum(c2_ref[pl.ds(b0 + 18, 7, stride=2), :],
                                c2_ref[pl.ds(b0 + 19, 7, stride=2), :]))
                blocks.append(pm)
        pooled = jnp.maximum(jnp.concatenate(blocks, axis=0) + b2_ref[...],
                             0.0).astype(jnp.bfloat16)

        p_id = lax.broadcasted_iota(jnp.int32, (49, 490), 0) * 10
        q_id = lax.broadcasted_iota(jnp.int32, (49, 490), 1)
        msk = (q_id >= p_id) & (q_id < p_id + 10)
        srows = []
        for g in range(G):
            y = jnp.dot(pooled[g * 49:(g + 1) * 49, :], wc_ref[...],
                        preferred_element_type=jnp.float32)
            srows.append(jnp.sum(jnp.where(msk, y, 0.0), axis=0,
                                 keepdims=True))
        s = jnp.concatenate(srows, axis=0)
        o_ref[...] = jnp.dot(s, rf_ref[...],
                             preferred_element_type=jnp.float32) + bl_ref[...]
    return body


def _patches(x_nchw):
    B = x_nchw.shape[0]
    xp = jnp.pad(x_nchw[:, 0, :, :], ((0, 0), (2, 2), (2, 2)))
    groups = []
    for dh in range(2):
        for dw in range(2):
            taps = [xp[:, dh + kh::2, dw + kw::2][:, :14, :14]
                    for kh in range(5) for kw in range(5)]
            g = jnp.stack(taps, axis=-1)
            g = jnp.concatenate([g, jnp.ones_like(g[..., :1])], axis=-1)
            g = jnp.pad(g, ((0, 0), (2, 2), (2, 2), (0, 0)))
            groups.append(g.reshape(B, 324, 26))
    p1 = jnp.concatenate(groups, axis=-1)
    p1 = jnp.pad(p1, ((0, 0), (0, _S1 - 324), (0, 0)))
    return p1.reshape(B * _S1, 104).astype(jnp.bfloat16)


def kernel(x, w1bd, w2m, b2m, wc, rf, blm):
    B = x.shape[0]
    G = 8 if B % 8 == 0 else 1
    SG = G * _S1
    C2G = SG - 80
    p1 = _patches(x)

    flops = B * (2 * _S1 * 104 * 64 + 2 * 248 * 400 * 32
                 + 2 * 49 * 32 * 490) + (B // G) * 2 * G * 490 * 10
    bytes_accessed = (p1.size * 2 + (w1bd.size + w2m.size + wc.size) * 2
                      + (b2m.size + rf.size + blm.size) * 4 + B * 10 * 4)

    return pl.pallas_call(
        _fused_body(G, SG, C2G),
        out_shape=jax.ShapeDtypeStruct((B, 10), jnp.float32),
        grid=(B // G,),
        in_specs=[
            pl.BlockSpec((SG, 104), lambda b: (b, 0)),
            pl.BlockSpec((104, 64), lambda b: (0, 0)),
            pl.BlockSpec((400, 32), lambda b: (0, 0)),
            pl.BlockSpec((1, 32), lambda b: (0, 0)),
            pl.BlockSpec((32, 490), lambda b: (0, 0)),
            pl.BlockSpec((490, 10), lambda b: (0, 0)),
            pl.BlockSpec((1, 10), lambda b: (0, 0)),
        ],
        out_specs=pl.BlockSpec((G, 10), lambda b: (b, 0)),
        scratch_shapes=[
            pltpu.VMEM((SG, 16), jnp.bfloat16),
            pltpu.VMEM((C2G, 32), jnp.float32),
        ],
        compiler_params=pltpu.CompilerParams(
            dimension_semantics=("parallel",)),
        cost_estimate=pl.CostEstimate(flops=int(flops), transcendentals=0,
                                      bytes_accessed=int(bytes_accessed)),
    )(p1, w1bd, w2m, b2m, wc, rf, blm)


# speedup vs baseline: 6.0304x; 3.6894x over previous
import numpy as np

import jax
import jax.numpy as jnp
from jax import lax
from jax.experimental import pallas as pl
from jax.experimental.pallas import tpu as pltpu


def _build_consts():
    s1 = np.zeros((5, 32, 512), np.float32)
    for kw in range(5):
        for j in range(14):
            for dw in range(2):
                u = 2 * j + dw + kw
                if u < 32:
                    s1[kw, u, dw * 256 + j * 16:dw * 256 + j * 16 + 16] = 1.0
    s2 = np.zeros((5, 288, 512), np.float32)
    for kw in range(5):
        for j2 in range(7):
            for dw2 in range(2):
                w1 = 2 * j2 + dw2 + kw
                s2[kw, w1 * 16:w1 * 16 + 16,
                   dw2 * 256 + j2 * 32:dw2 * 256 + j2 * 32 + 32] = 1.0
    r = np.arange(256)
    q = np.arange(70)
    c2i = np.broadcast_to((r % 32)[:, None], (256, 70))
    pi = ((q[None, :] // 10) * 7 + (r // 32)[:, None]) * 10 + q[None, :] % 10
    rowok = (r // 32 < 7)[:, None]
    rfb = (np.arange(70)[:, None] % 10 == np.arange(10)[None, :])
    return (jnp.asarray(s1), jnp.asarray(s2), jnp.asarray(c2i),
            jnp.asarray(np.where(rowok, pi, 0)), jnp.asarray(rowok),
            jnp.asarray(rfb.astype(np.float32)))


_S1M, _S2M, _C2I, _PI, _ROWOK, _RFB = _build_consts()


def _body(G, out8):
    RS = G * 40
    R2 = G * 18

    def body(xb_ref, w1_ref, w2_ref, b1_ref, b2_ref, wcb_ref, rfb_ref,
             bl_ref, o_ref, z1_ref, a1_ref, c2_ref):
        for r0 in range(0, RS - 8, 104):
            rs = min(104, RS - 8 - r0)
            acc = None
            for kh in range(5):
                p = jnp.dot(xb_ref[r0 + kh:r0 + kh + rs, :],
                            w1_ref[kh * 32:(kh + 1) * 32, :],
                            preferred_element_type=jnp.float32)
                acc = p if acc is None else acc + p
            for k in range(4):
                z1_ref[k, r0:r0 + rs, :] = acc[:, 128 * k:128 * (k + 1)]

        a1_ref[...] = jnp.zeros((R2 + 8, 288), jnp.bfloat16)
        for g in range(G):
            vh = [jnp.maximum(z1_ref[k, pl.ds(g * 40, 14, stride=2), :],
                              z1_ref[k, pl.ds(g * 40 + 1, 14, stride=2), :])
                  for k in range(4)]
            v = jnp.concatenate([jnp.maximum(vh[0], vh[2]),
                                 jnp.maximum(vh[1], vh[3])], axis=1)
            a1_ref[g * 18 + 2:g * 18 + 16, 32:288] = jnp.maximum(
                v + b1_ref[...], 0.0).astype(jnp.bfloat16)

        for r0 in range(0, R2, 72):
            rs = min(72, R2 - r0)
            acc = None
            for kh in range(5):
                p = jnp.dot(a1_ref[r0 + kh:r0 + kh + rs, :],
                            w2_ref[kh * 288:(kh + 1) * 288, :],
                            preferred_element_type=jnp.float32)
                acc = p if acc is None else acc + p
            for k in range(4):
                c2_ref[k, r0:r0 + rs, :] = acc[:, 128 * k:128 * (k + 1)]

        rows = []
        for g in range(G):
            vh = [jnp.maximum(c2_ref[k, pl.ds(g * 18, 7, stride=2), :],
                              c2_ref[k, pl.ds(g * 18 + 1, 7, stride=2), :])
                  for k in range(4)]
            v = jnp.concatenate([jnp.maximum(vh[0], vh[2]),
                                 jnp.maximum(vh[1], vh[3])], axis=1)
            rows.append(jnp.maximum(v + b2_ref[...], 0.0))
        pp = jnp.concatenate(rows, axis=0).astype(jnp.bfloat16)

        y = jnp.dot(pp, wcb_ref[...], preferred_element_type=jnp.float32)
        r_id = lax.broadcasted_iota(jnp.int32, (G * 7, 70), 0) % 7
        q_id = lax.broadcasted_iota(jnp.int32, (G * 7, 70), 1)
        y = jnp.where((q_id >= r_id * 10) & (q_id < r_id * 10 + 10), y, 0.0)
        s = jnp.dot(y, rfb_ref[...], preferred_element_type=jnp.float32)
        g_id = lax.broadcasted_iota(jnp.int32, (G, G * 7), 0)
        c_id = lax.broadcasted_iota(jnp.int32, (G, G * 7), 1)
        ss = (c_id // 7 == g_id).astype(jnp.float32)
        out = jnp.dot(ss, s, preferred_element_type=jnp.float32) + bl_ref[...]
        if out8:
            o_ref[...] = jnp.broadcast_to(out, (8, 10))
        else:
            o_ref[...] = out
    return body


def kernel(x, w1bd, w2m, b2m, wc, rf, blm):
    B = x.shape[0]
    G = 8 if B % 8 == 0 else 1
    out8 = G == 1

    w1m = w1bd[:26, :16].astype(jnp.float32)
    w1s = []
    for kh in range(5):
        acc = 0.0
        for kw in range(5):
            row = jnp.broadcast_to(jnp.tile(w1m[kh * 5 + kw], 32), (32, 512))
            acc = acc + _S1M[kw] * row
        w1s.append(acc)
    w1s = jnp.concatenate(w1s, axis=0).astype(jnp.bfloat16)
    w2f = w2m.astype(jnp.float32)
    w2s = []
    for kh in range(5):
        acc = 0.0
        for kw in range(5):
            blk = jnp.tile(w2f[(kh * 5 + kw) * 16:(kh * 5 + kw) * 16 + 16, :],
                           (18, 16))
            acc = acc + _S2M[kw] * blk
        w2s.append(acc)
    w2s = jnp.concatenate(w2s, axis=0).astype(jnp.bfloat16)
    b1r = jnp.concatenate([jnp.tile(w1m[25], 14),
                           jnp.zeros(32, jnp.float32)]).reshape(1, 256)
    b2r = jnp.concatenate([jnp.tile(b2m, (1, 7)),
                           jnp.zeros((1, 32), jnp.float32)], axis=1)
    wcb = jnp.where(_ROWOK, wc.astype(jnp.float32)[_C2I, _PI],
                    0.0).astype(jnp.bfloat16)

    xb = jnp.pad(x[:, 0, :, :], ((0, 0), (2, 10), (2, 2)))
    xb = xb.astype(jnp.bfloat16).reshape(B * 40, 32)

    flops = B * 2 * (28 * 32 * 512 + 14 * 288 * 512 + 7 * 256 * 70)
    bytes_accessed = int(xb.size * 2 + w1s.size * 2 + w2s.size * 2
                         + wcb.size * 2 + B * 10 * 4)

    out = pl.pallas_call(
        _body(G, out8),
        out_shape=jax.ShapeDtypeStruct((B * 8 if out8 else B, 10), jnp.float32),
        grid=(B // G,),
        in_specs=[
            pl.BlockSpec((G * 40, 32), lambda b: (b, 0)),
            pl.BlockSpec((160, 512), lambda b: (0, 0)),
            pl.BlockSpec((1440, 512), lambda b: (0, 0)),
            pl.BlockSpec((1, 256), lambda b: (0, 0)),
            pl.BlockSpec((1, 256), lambda b: (0, 0)),
            pl.BlockSpec((256, 70), lambda b: (0, 0)),
            pl.BlockSpec((70, 10), lambda b: (0, 0)),
            pl.BlockSpec((1, 10), lambda b: (0, 0)),
        ],
        out_specs=pl.BlockSpec((8, 10) if out8 else (G, 10), lambda b: (b, 0)),
        scratch_shapes=[
            pltpu.VMEM((4, G * 40, 128), jnp.float32),
            pltpu.VMEM((G * 18 + 8, 288), jnp.bfloat16),
            pltpu.VMEM((4, G * 18, 128), jnp.float32),
        ],
        compiler_params=pltpu.CompilerParams(
            dimension_semantics=("parallel",)),
        cost_estimate=pl.CostEstimate(flops=int(flops), transcendentals=0,
                                      bytes_accessed=bytes_accessed),
    )(xb, w1s, w2s, b1r, b2r, wcb, _RFB, blm)
    return out[0::8, :] if out8 else out


# speedup vs baseline: 7.3974x; 1.2267x over previous
import numpy as np

import jax
import jax.numpy as jnp
from jax import lax
from jax.experimental import pallas as pl
from jax.experimental.pallas import tpu as pltpu


def _build_consts():
    s1 = np.zeros((5, 32, 512), np.float32)
    for kw in range(5):
        for j in range(14):
            for dw in range(2):
                u = 2 * j + dw + kw
                if u < 32:
                    s1[kw, u, dw * 256 + j * 16:dw * 256 + j * 16 + 16] = 1.0
    s2 = np.zeros((5, 288, 512), np.float32)
    for kw in range(5):
        for j2 in range(7):
            for dw2 in range(2):
                w1 = 2 * j2 + dw2 + kw
                s2[kw, w1 * 16:w1 * 16 + 16,
                   dw2 * 256 + j2 * 32:dw2 * 256 + j2 * 32 + 32] = 1.0
    r = np.arange(256)
    q = np.arange(70)
    c2i = np.broadcast_to((r % 32)[:, None], (256, 70))
    pi = ((q[None, :] // 10) * 7 + (r // 32)[:, None]) * 10 + q[None, :] % 10
    rowok = (r // 32 < 7)[:, None]
    rfb = (np.arange(70)[:, None] % 10 == np.arange(10)[None, :])
    return (s1, s2, c2i, np.where(rowok, pi, 0), rowok,
            rfb.astype(np.float32))


_S1M, _S2M, _C2I, _PI, _ROWOK, _RFB = _build_consts()


def _body(G, out8):
    RS = G * 40
    R2 = G * 18

    def body(xb_ref, w1_ref, w2_ref, b1_ref, b2_ref, wcb_ref, rfb_ref,
             bl_ref, o_ref, z1_ref, a1_ref, c2_ref):
        for r0 in range(0, RS - 8, 104):
            rs = min(104, RS - 8 - r0)
            acc = None
            for kh in range(5):
                p = jnp.dot(xb_ref[r0 + kh:r0 + kh + rs, :],
                            w1_ref[kh * 32:(kh + 1) * 32, :],
                            preferred_element_type=jnp.float32)
                acc = p if acc is None else acc + p
            for k in range(4):
                z1_ref[k, r0:r0 + rs, :] = acc[:, 128 * k:128 * (k + 1)]

        a1_ref[...] = jnp.zeros((R2 + 8, 288), jnp.bfloat16)
        for g in range(G):
            vh = [jnp.maximum(z1_ref[k, pl.ds(g * 40, 14, stride=2), :],
                              z1_ref[k, pl.ds(g * 40 + 1, 14, stride=2), :])
                  for k in range(4)]
            v = jnp.concatenate([jnp.maximum(vh[0], vh[2]),
                                 jnp.maximum(vh[1], vh[3])], axis=1)
            a1_ref[g * 18 + 2:g * 18 + 16, 32:288] = jnp.maximum(
                v + b1_ref[...], 0.0).astype(jnp.bfloat16)

        for r0 in range(0, R2, 72):
            rs = min(72, R2 - r0)
            acc = None
            for kh in range(5):
                p = jnp.dot(a1_ref[r0 + kh:r0 + kh + rs, :],
                            w2_ref[kh * 288:(kh + 1) * 288, :],
                            preferred_element_type=jnp.float32)
                acc = p if acc is None else acc + p
            for k in range(4):
                c2_ref[k, r0:r0 + rs, :] = acc[:, 128 * k:128 * (k + 1)]

        rows = []
        for g in range(G):
            vh = [jnp.maximum(c2_ref[k, pl.ds(g * 18, 7, stride=2), :],
                              c2_ref[k, pl.ds(g * 18 + 1, 7, stride=2), :])
                  for k in range(4)]
            v = jnp.concatenate([jnp.maximum(vh[0], vh[2]),
                                 jnp.maximum(vh[1], vh[3])], axis=1)
            rows.append(jnp.maximum(v + b2_ref[...], 0.0))
        pp = jnp.concatenate(rows, axis=0).astype(jnp.bfloat16)

        y = jnp.dot(pp, wcb_ref[...], preferred_element_type=jnp.float32)
        r_id = lax.broadcasted_iota(jnp.int32, (G * 7, 70), 0) % 7
        q_id = lax.broadcasted_iota(jnp.int32, (G * 7, 70), 1)
        y = jnp.where((q_id >= r_id * 10) & (q_id < r_id * 10 + 10), y, 0.0)
        s = jnp.dot(y, rfb_ref[...], preferred_element_type=jnp.float32)
        g_id = lax.broadcasted_iota(jnp.int32, (G, G * 7), 0)
        c_id = lax.broadcasted_iota(jnp.int32, (G, G * 7), 1)
        ss = (c_id // 7 == g_id).astype(jnp.float32)
        out = jnp.dot(ss, s, preferred_element_type=jnp.float32) + bl_ref[...]
        if out8:
            o_ref[...] = jnp.broadcast_to(out, (8, 10))
        else:
            o_ref[...] = out
    return body


def kernel(x, w1bd, w2m, b2m, wc, rf, blm):
    B = x.shape[0]
    G = 32 if B % 32 == 0 else (8 if B % 8 == 0 else 1)
    out8 = G == 1

    w1m = w1bd[:26, :16].astype(jnp.float32)
    w1s = []
    for kh in range(5):
        acc = 0.0
        for kw in range(5):
            row = jnp.broadcast_to(jnp.tile(w1m[kh * 5 + kw], 32), (32, 512))
            acc = acc + _S1M[kw] * row
        w1s.append(acc)
    w1s = jnp.concatenate(w1s, axis=0).astype(jnp.bfloat16)
    w2f = w2m.astype(jnp.float32)
    w2s = []
    for kh in range(5):
        acc = 0.0
        for kw in range(5):
            blk = jnp.tile(w2f[(kh * 5 + kw) * 16:(kh * 5 + kw) * 16 + 16, :],
                           (18, 16))
            acc = acc + _S2M[kw] * blk
        w2s.append(acc)
    w2s = jnp.concatenate(w2s, axis=0).astype(jnp.bfloat16)
    b1r = jnp.concatenate([jnp.tile(w1m[25], 14),
                           jnp.zeros(32, jnp.float32)]).reshape(1, 256)
    b2r = jnp.concatenate([jnp.tile(b2m, (1, 7)),
                           jnp.zeros((1, 32), jnp.float32)], axis=1)
    wcb = jnp.where(_ROWOK, wc.astype(jnp.float32)[_C2I, _PI],
                    0.0).astype(jnp.bfloat16)

    xb = jnp.pad(x[:, 0, :, :], ((0, 0), (2, 10), (2, 2)))
    xb = xb.astype(jnp.bfloat16).reshape(B * 40, 32)

    flops = B * 2 * (28 * 32 * 512 + 14 * 288 * 512 + 7 * 256 * 70)
    bytes_accessed = int(xb.size * 2 + w1s.size * 2 + w2s.size * 2
                         + wcb.size * 2 + B * 10 * 4)

    out = pl.pallas_call(
        _body(G, out8),
        out_shape=jax.ShapeDtypeStruct((B * 8 if out8 else B, 10), jnp.float32),
        grid=(B // G,),
        in_specs=[
            pl.BlockSpec((G * 40, 32), lambda b: (b, 0)),
            pl.BlockSpec((160, 512), lambda b: (0, 0)),
            pl.BlockSpec((1440, 512), lambda b: (0, 0)),
            pl.BlockSpec((1, 256), lambda b: (0, 0)),
            pl.BlockSpec((1, 256), lambda b: (0, 0)),
            pl.BlockSpec((256, 70), lambda b: (0, 0)),
            pl.BlockSpec((70, 10), lambda b: (0, 0)),
            pl.BlockSpec((1, 10), lambda b: (0, 0)),
        ],
        out_specs=pl.BlockSpec((8, 10) if out8 else (G, 10), lambda b: (b, 0)),
        scratch_shapes=[
            pltpu.VMEM((4, G * 40, 128), jnp.float32),
            pltpu.VMEM((G * 18 + 8, 288), jnp.bfloat16),
            pltpu.VMEM((4, G * 18, 128), jnp.float32),
        ],
        compiler_params=pltpu.CompilerParams(
            dimension_semantics=("parallel",)),
        cost_estimate=pl.CostEstimate(flops=int(flops), transcendentals=0,
                                      bytes_accessed=bytes_accessed),
    )(xb, w1s, w2s, b1r, b2r, wcb, _RFB, blm)
    return out[0::8, :] if out8 else out
